# baseline (speedup 1.0000x reference)
import functools

import jax
import jax.numpy as jnp
from jax import lax
from jax.experimental import pallas as pl
from jax.experimental.pallas import tpu as pltpu
from jax.experimental.pallas import tpu_sc as plsc

_SIZE = 512
_LENGTH = 16
_ATT = 256
_R = 1024
_LANES = 16
_TILES = 32


def _make_step(bsz):
    rows = bsz // _TILES
    mesh = plsc.VectorSubcoreMesh(core_axis_name="c", subcore_axis_name="s")

    @functools.partial(
        pl.kernel,
        out_type=jax.ShapeDtypeStruct((bsz, _SIZE), jnp.float32),
        mesh=mesh,
        compiler_params=pltpu.CompilerParams(use_tc_tiling_on_sc=False, needs_layout_passes=False),
        scratch_types=[
            pltpu.VMEM((_R,), jnp.int32),
            pltpu.VMEM((_R,), jnp.int32),
            pltpu.VMEM((rows, _SIZE), jnp.float32),
            pltpu.VMEM((rows, _R), jnp.float32),
            pltpu.VMEM((rows, _SIZE), jnp.float32),
        ],
    )
    def step(subj_hbm, obj_hbm, state_hbm, h_hbm, out_hbm,
             subj_v, obj_v, state_v, h_v, out_v):
        wid = lax.axis_index("s") * 2 + lax.axis_index("c")
        base = wid * rows
        pltpu.sync_copy(subj_hbm, subj_v)
        pltpu.sync_copy(obj_hbm, obj_v)
        pltpu.sync_copy(state_hbm.at[pl.ds(base, rows)], state_v)
        pltpu.sync_copy(h_hbm.at[pl.ds(base, rows)], h_v)
        zeros = jnp.zeros((_LANES,), jnp.float32)
        for r in range(rows):
            for k in range(_SIZE // _LANES):
                out_v[r, pl.ds(k * _LANES, _LANES)] = zeros
        for r in range(rows):
            row_idx = jnp.full((_LANES,), r, jnp.int32)

            def body(k, carry):
                idx_s = subj_v[pl.ds(k * _LANES, _LANES)]
                idx_o = obj_v[pl.ds(k * _LANES, _LANES)]
                vals = plsc.load_gather(state_v, [row_idx, idx_s])
                hv = h_v[r, pl.ds(k * _LANES, _LANES)]
                plsc.addupdate_scatter(out_v, [row_idx, idx_o], vals * hv)
                return carry

            lax.fori_loop(0, _R // _LANES, body, 0)
        pltpu.sync_copy(out_v, out_hbm.at[pl.ds(base, rows)])

    return step


def kernel(x, rel_subj, rel_obj, rel_enc, Wrel, brel, action_table, pos_table,
           metaMode_init, W1G, b1G, W2G, b2G):
    bsz = x.shape[0]
    step = _make_step(bsz)

    state = x[:, :_SIZE].astype(jnp.float32)
    metaMode = jnp.broadcast_to(metaMode_init[None], (bsz, _ATT))
    relation = jnp.dot(rel_enc[:_R], Wrel) + brel
    outs = []
    for _ in range(_LENGTH):
        g_in = jnp.concatenate((state, metaMode), axis=1)
        metaMode = jax.nn.relu(jnp.dot(g_in, W1G) + b1G)
        metaMode = jnp.dot(metaMode, W2G) + b2G
        h = jax.nn.sigmoid(jnp.dot(metaMode, relation.T))
        state = step(rel_subj, rel_obj, state, h)
        outs.append(state)
    return jnp.stack(outs, axis=1)


# speedup vs baseline: 1.1500x; 1.1500x over previous
import functools

import jax
import jax.numpy as jnp
from jax import lax
from jax.experimental import pallas as pl
from jax.experimental.pallas import tpu as pltpu
from jax.experimental.pallas import tpu_sc as plsc

_SIZE = 512
_LENGTH = 16
_ATT = 256
_R = 1024
_LANES = 16
_TILES = 32


def _make_step(bsz):
    rows = bsz // _TILES
    mesh = plsc.VectorSubcoreMesh(core_axis_name="c", subcore_axis_name="s")

    @functools.partial(
        pl.kernel,
        out_type=jax.ShapeDtypeStruct((bsz, _SIZE), jnp.float32),
        mesh=mesh,
        compiler_params=pltpu.CompilerParams(use_tc_tiling_on_sc=False, needs_layout_passes=False),
        scratch_types=[
            pltpu.VMEM((_R,), jnp.int32),
            pltpu.VMEM((_R,), jnp.int32),
            pltpu.VMEM((rows, _SIZE), jnp.float32),
            pltpu.VMEM((rows, _R), jnp.float32),
            pltpu.VMEM((rows, _SIZE), jnp.float32),
        ],
    )
    def step(subj_hbm, obj_hbm, state_hbm, h_hbm, out_hbm,
             subj_v, obj_v, state_v, h_v, out_v):
        wid = lax.axis_index("s") * 2 + lax.axis_index("c")
        base = wid * rows
        pltpu.sync_copy(subj_hbm, subj_v)
        pltpu.sync_copy(obj_hbm, obj_v)
        pltpu.sync_copy(state_hbm.at[pl.ds(base, rows)], state_v)
        pltpu.sync_copy(h_hbm.at[pl.ds(base, rows)], h_v)
        zeros = jnp.zeros((_LANES,), jnp.float32)
        for r in range(rows):
            for k in range(_SIZE // _LANES):
                out_v[r, pl.ds(k * _LANES, _LANES)] = zeros
        row_ids = [jnp.full((_LANES,), r, jnp.int32) for r in range(rows)]

        def body(k, carry):
            idx_s = subj_v[pl.ds(k * _LANES, _LANES)]
            idx_o = obj_v[pl.ds(k * _LANES, _LANES)]
            for r in range(rows):
                vals = plsc.load_gather(state_v, [row_ids[r], idx_s])
                hv = h_v[r, pl.ds(k * _LANES, _LANES)]
                plsc.addupdate_scatter(out_v, [row_ids[r], idx_o], vals * hv)
            return carry

        lax.fori_loop(0, _R // _LANES, body, 0, unroll=2)
        pltpu.sync_copy(out_v, out_hbm.at[pl.ds(base, rows)])

    return step


def kernel(x, rel_subj, rel_obj, rel_enc, Wrel, brel, action_table, pos_table,
           metaMode_init, W1G, b1G, W2G, b2G):
    bsz = x.shape[0]
    step = _make_step(bsz)

    state = x[:, :_SIZE].astype(jnp.float32)
    metaMode = jnp.broadcast_to(metaMode_init[None], (bsz, _ATT))
    relation = jnp.dot(rel_enc[:_R], Wrel) + brel
    outs = []
    for _ in range(_LENGTH):
        g_in = jnp.concatenate((state, metaMode), axis=1)
        metaMode = jax.nn.relu(jnp.dot(g_in, W1G) + b1G)
        metaMode = jnp.dot(metaMode, W2G) + b2G
        h = jax.nn.sigmoid(jnp.dot(metaMode, relation.T))
        state = step(rel_subj, rel_obj, state, h)
        outs.append(state)
    return jnp.stack(outs, axis=1)
